# baseline (speedup 1.0000x reference)
import functools

import jax
import jax.numpy as jnp
from jax import lax
from jax.experimental import pallas as pl
from jax.experimental.pallas import tpu as pltpu
from jax.experimental.pallas import tpu_sc as plsc

D = 64
V = 360
BATCH = 16384
HIST = 200

NC = 2
NS = 16
NW = NC * NS

B_PER_W = BATCH // NW
BT_PER_W = B_PER_W // 128
NJ = B_PER_W // 16


def _table_transform_body(tbl_ref, w_ref, b_ref, out_ref):
    prod = lax.dot_general(
        w_ref[...], tbl_ref[...],
        (((1,), (1,)), ((), ())),
        preferred_element_type=jnp.float32,
    )
    out_ref[...] = jnp.maximum(prod + b_ref[...], 0.0)


def _table_transform(table, W, b2):
    return pl.pallas_call(
        _table_transform_body,
        out_shape=jax.ShapeDtypeStruct((D, V), jnp.float32),
    )(table, W, b2)


_MESH = plsc.VectorSubcoreMesh(core_axis_name="c", subcore_axis_name="s")


@functools.partial(
    pl.kernel,
    out_type=jax.ShapeDtypeStruct((HIST, D // 8, 128, 8, 128), jnp.float32),
    mesh=_MESH,
    compiler_params=pltpu.CompilerParams(
        use_tc_tiling_on_sc=False, needs_layout_passes=False),
    scratch_types=[
        pltpu.VMEM((V * D,), jnp.float32),
        pltpu.VMEM((B_PER_W,), jnp.int32),
        pltpu.VMEM((B_PER_W,), jnp.int32),
        pltpu.VMEM((D // 8, BT_PER_W, 8, 128), jnp.float32),
        pltpu.VMEM((D // 8, BT_PER_W, 8, 128), jnp.float32),
        pltpu.SemaphoreType.DMA,
        pltpu.SemaphoreType.DMA,
        pltpu.SemaphoreType.DMA,
        pltpu.SemaphoreType.DMA,
    ],
)
def _gather_kernel(idxt_hbm, t2_hbm, out_hbm, t2_v,
                   idx_v0, idx_v1, buf0, buf1,
                   sem_t2, sem_idx, sem_st0, sem_st1):
    cid = lax.axis_index("c")
    sid = lax.axis_index("s")
    wid = sid * NC + cid
    bt0 = wid * BT_PER_W

    pltpu.async_copy(t2_hbm, t2_v, sem_t2).wait()

    idx_bufs = (idx_v0, idx_v1)
    bufs = (buf0, buf1)
    st_sems = (sem_st0, sem_st1)

    pltpu.async_copy(
        idxt_hbm.at[0, pl.ds(wid * B_PER_W, B_PER_W)], idx_v0, sem_idx)

    def pair(p, carry):
        for half in range(2):
            h = 2 * p + half
            idx_v = idx_bufs[half]
            buf = bufs[half]
            sem_st = st_sems[half]

            pltpu.make_async_copy(
                idxt_hbm.at[0, pl.ds(wid * B_PER_W, B_PER_W)],
                idx_v, sem_idx).wait()

            @pl.when(h + 1 < HIST)
            def _():
                pltpu.async_copy(
                    idxt_hbm.at[h + 1, pl.ds(wid * B_PER_W, B_PER_W)],
                    idx_bufs[1 - half], sem_idx)

            @pl.when(p > 0)
            def _():
                for dt in range(D // 8):
                    pltpu.make_async_copy(
                        buf.at[dt], out_hbm.at[0, dt, pl.ds(0, BT_PER_W)],
                        sem_st).wait()

            @plsc.parallel_loop(0, NJ, step=1, unroll=8)
            def _(j):
                jj = j // 8
                jo = j % 8
                v = idx_v[pl.ds(j * 16, 16)]
                for dt in range(D // 8):
                    for ds in range(8):
                        g = plsc.load_gather(t2_v, [v + (dt * 8 + ds) * V])
                        buf[dt, jj, ds, pl.ds(jo * 16, 16)] = g

            for dt in range(D // 8):
                pltpu.async_copy(
                    buf.at[dt],
                    out_hbm.at[h, dt, pl.ds(bt0, BT_PER_W)],
                    sem_st)
        return carry

    lax.fori_loop(0, HIST // 2, pair, 0)

    for half in range(2):
        for dt in range(D // 8):
            pltpu.make_async_copy(
                bufs[half].at[dt],
                out_hbm.at[0, dt, pl.ds(0, BT_PER_W)],
                st_sems[half]).wait()


def kernel(angles, table, W, b):
    t2 = _table_transform(table, W, b.reshape(D, 1)).reshape(D * V)
    idx_t = jnp.transpose(angles.astype(jnp.int32))
    out5 = _gather_kernel(idx_t, t2)
    out = out5.transpose(0, 1, 3, 2, 4).reshape(HIST, D, BATCH)
    return out.transpose(2, 0, 1)


# speedup vs baseline: 1.1249x; 1.1249x over previous
import functools

import jax
import jax.numpy as jnp
from jax import lax
from jax.experimental import pallas as pl
from jax.experimental.pallas import tpu as pltpu
from jax.experimental.pallas import tpu_sc as plsc

D = 64
V = 360
BATCH = 16384
HIST = 200

NC = 2
NS = 16
NW = NC * NS

B_PER_W = BATCH // NW
BT_PER_W = B_PER_W // 128
NJ = B_PER_W // 16


def _table_transform_body(tbl_ref, w_ref, b_ref, out_ref):
    prod = lax.dot_general(
        w_ref[...], tbl_ref[...],
        (((1,), (1,)), ((), ())),
        preferred_element_type=jnp.float32,
    )
    out_ref[...] = jnp.maximum(prod + b_ref[...], 0.0)


def _table_transform(table, W, b2):
    return pl.pallas_call(
        _table_transform_body,
        out_shape=jax.ShapeDtypeStruct((D, V), jnp.float32),
    )(table, W, b2)


_MESH = plsc.VectorSubcoreMesh(core_axis_name="c", subcore_axis_name="s")


@functools.partial(
    pl.kernel,
    out_type=jax.ShapeDtypeStruct((HIST, D // 8, 128, 8, 128), jnp.float32),
    mesh=_MESH,
    compiler_params=pltpu.CompilerParams(
        use_tc_tiling_on_sc=False, needs_layout_passes=False),
    scratch_types=[
        pltpu.VMEM((V * D,), jnp.float32),
        pltpu.VMEM((B_PER_W,), jnp.int32),
        pltpu.VMEM((B_PER_W,), jnp.int32),
        pltpu.VMEM((D // 8, BT_PER_W, 8, 128), jnp.float32),
        pltpu.VMEM((D // 8, BT_PER_W, 8, 128), jnp.float32),
        pltpu.SemaphoreType.DMA,
        pltpu.SemaphoreType.DMA,
        pltpu.SemaphoreType.DMA,
        pltpu.SemaphoreType.DMA,
    ],
)
def _gather_kernel(idxt_hbm, t2_hbm, out_hbm, t2_v,
                   idx_v0, idx_v1, buf0, buf1,
                   sem_t2, sem_idx, sem_st0, sem_st1):
    cid = lax.axis_index("c")
    sid = lax.axis_index("s")
    wid = sid * NC + cid
    bt0 = wid * BT_PER_W

    pltpu.async_copy(t2_hbm, t2_v, sem_t2).wait()

    idx_bufs = (idx_v0, idx_v1)
    bufs = (buf0, buf1)
    st_sems = (sem_st0, sem_st1)

    pltpu.async_copy(
        idxt_hbm.at[0, pl.ds(wid * B_PER_W, B_PER_W)], idx_v0, sem_idx)

    def pair(p, carry):
        for half in range(2):
            h = 2 * p + half
            idx_v = idx_bufs[half]
            buf = bufs[half]
            sem_st = st_sems[half]

            pltpu.make_async_copy(
                idxt_hbm.at[0, pl.ds(wid * B_PER_W, B_PER_W)],
                idx_v, sem_idx).wait()

            @pl.when(h + 1 < HIST)
            def _():
                pltpu.async_copy(
                    idxt_hbm.at[h + 1, pl.ds(wid * B_PER_W, B_PER_W)],
                    idx_bufs[1 - half], sem_idx)

            @pl.when(p > 0)
            def _():
                for dt in range(D // 8):
                    pltpu.make_async_copy(
                        buf.at[dt], out_hbm.at[0, dt, pl.ds(0, BT_PER_W)],
                        sem_st).wait()

            @plsc.parallel_loop(0, NJ, step=1, unroll=4)
            def _(j):
                jj = j // 8
                jo = j % 8
                v = idx_v[pl.ds(j * 16, 16)]
                for dt in range(D // 8):
                    for ds in range(8):
                        g = plsc.load_gather(t2_v, [v + (dt * 8 + ds) * V])
                        buf[dt, jj, ds, pl.ds(jo * 16, 16)] = g

            for dt in range(D // 8):
                pltpu.async_copy(
                    buf.at[dt],
                    out_hbm.at[h, dt, pl.ds(bt0, BT_PER_W)],
                    sem_st)
        return carry

    lax.fori_loop(0, HIST // 2, pair, 0)

    for half in range(2):
        for dt in range(D // 8):
            pltpu.make_async_copy(
                bufs[half].at[dt],
                out_hbm.at[0, dt, pl.ds(0, BT_PER_W)],
                st_sems[half]).wait()


def kernel(angles, table, W, b):
    t2 = _table_transform(table, W, b.reshape(D, 1)).reshape(D * V)
    idx_t = jnp.transpose(angles.astype(jnp.int32))
    out5 = _gather_kernel(idx_t, t2)
    out = out5.transpose(0, 1, 3, 2, 4).reshape(HIST, D, BATCH)
    return out.transpose(2, 0, 1)


# speedup vs baseline: 1.1386x; 1.0121x over previous
import functools

import jax
import jax.numpy as jnp
from jax import lax
from jax.experimental import pallas as pl
from jax.experimental.pallas import tpu as pltpu
from jax.experimental.pallas import tpu_sc as plsc

D = 64
V = 360
BATCH = 16384
HIST = 200

NC = 2
NS = 16
NW = NC * NS

B_PER_W = BATCH // NW
BT_PER_W = B_PER_W // 128
NJ = B_PER_W // 16


def _table_transform_body(tbl_ref, w_ref, b_ref, out_ref):
    prod = lax.dot_general(
        w_ref[...], tbl_ref[...],
        (((1,), (1,)), ((), ())),
        preferred_element_type=jnp.float32,
    )
    out_ref[...] = jnp.maximum(prod + b_ref[...], 0.0)


def _table_transform(table, W, b2):
    return pl.pallas_call(
        _table_transform_body,
        out_shape=jax.ShapeDtypeStruct((D, V), jnp.float32),
    )(table, W, b2)


_MESH = plsc.VectorSubcoreMesh(core_axis_name="c", subcore_axis_name="s")


@functools.partial(
    pl.kernel,
    out_type=jax.ShapeDtypeStruct((HIST, D // 8, 128, 8, 128), jnp.float32),
    mesh=_MESH,
    compiler_params=pltpu.CompilerParams(
        use_tc_tiling_on_sc=False, needs_layout_passes=False),
    scratch_types=[
        pltpu.VMEM((V * D,), jnp.float32),
        pltpu.VMEM((B_PER_W,), jnp.int32),
        pltpu.VMEM((B_PER_W,), jnp.int32),
        pltpu.VMEM((D // 8, BT_PER_W, 8, 128), jnp.float32),
        pltpu.VMEM((D // 8, BT_PER_W, 8, 128), jnp.float32),
        pltpu.SemaphoreType.DMA,
        pltpu.SemaphoreType.DMA,
        pltpu.SemaphoreType.DMA,
        pltpu.SemaphoreType.DMA,
    ],
)
def _gather_kernel(idxt_hbm, t2_hbm, out_hbm, t2_v,
                   idx_v0, idx_v1, buf0, buf1,
                   sem_t2, sem_idx, sem_st0, sem_st1):
    cid = lax.axis_index("c")
    sid = lax.axis_index("s")
    wid = sid * NC + cid
    bt0 = wid * BT_PER_W

    pltpu.async_copy(t2_hbm, t2_v, sem_t2).wait()

    idx_bufs = (idx_v0, idx_v1)
    bufs = (buf0, buf1)
    st_sems = (sem_st0, sem_st1)

    pltpu.async_copy(
        idxt_hbm.at[0, pl.ds(wid * B_PER_W, B_PER_W)], idx_v0, sem_idx)

    def pair(p, carry):
        for half in range(2):
            h = 2 * p + half
            idx_v = idx_bufs[half]
            buf = bufs[half]
            sem_st = st_sems[half]

            pltpu.make_async_copy(
                idxt_hbm.at[0, pl.ds(wid * B_PER_W, B_PER_W)],
                idx_v, sem_idx).wait()

            @pl.when(h + 1 < HIST)
            def _():
                pltpu.async_copy(
                    idxt_hbm.at[h + 1, pl.ds(wid * B_PER_W, B_PER_W)],
                    idx_bufs[1 - half], sem_idx)

            @pl.when(p > 0)
            def _():
                pltpu.make_async_copy(
                    buf, out_hbm.at[0, :, pl.ds(0, BT_PER_W)],
                    sem_st).wait()

            @plsc.parallel_loop(0, NJ, step=1, unroll=4)
            def _(j):
                jj = j // 8
                jo = j % 8
                v = idx_v[pl.ds(j * 16, 16)]
                for dt in range(D // 8):
                    for ds in range(8):
                        g = plsc.load_gather(t2_v, [v + (dt * 8 + ds) * V])
                        buf[dt, jj, ds, pl.ds(jo * 16, 16)] = g

            pltpu.async_copy(
                buf, out_hbm.at[h, :, pl.ds(bt0, BT_PER_W)], sem_st)
        return carry

    lax.fori_loop(0, HIST // 2, pair, 0)

    for half in range(2):
        pltpu.make_async_copy(
            bufs[half], out_hbm.at[0, :, pl.ds(0, BT_PER_W)],
            st_sems[half]).wait()


def kernel(angles, table, W, b):
    t2 = _table_transform(table, W, b.reshape(D, 1)).reshape(D * V)
    idx_t = jnp.transpose(angles.astype(jnp.int32))
    out5 = _gather_kernel(idx_t, t2)
    out = out5.transpose(0, 1, 3, 2, 4).reshape(HIST, D, BATCH)
    return out.transpose(2, 0, 1)
